# baseline (speedup 1.0000x reference)
import functools
import math

import jax
import jax.numpy as jnp
from jax import lax
from jax.experimental import pallas as pl
from jax.experimental.pallas import tpu as pltpu
from jax.experimental.pallas import tpu_sc as plsc

N = 10000
E = 320000
D = 128
L = 3

NC = 2
NS = 16
TILES = NC * NS
EPT = E // TILES
CH = 80
NCHUNK = EPT // CH
IB = 25
NG = NCHUNK // IB
K = 3
ACC_R = N
WPS = 640
ZR = 16

_mesh = plsc.VectorSubcoreMesh(
    core_axis_name="c", subcore_axis_name="s", num_cores=NC, num_subcores=NS)

_agg_out_type = jax.ShapeDtypeStruct((NC, N, D), jnp.float32)


def _zero_acc(acc, zbuf, s):
    def zstore(t, carry):
        i = t // (D // 16)
        k = t % (D // 16)
        zbuf[i, pl.ds(k * 16, 16)] = jnp.zeros((16,), jnp.float32)
        return carry
    lax.fori_loop(0, ZR * (D // 16), zstore, 0)
    z = jnp.where(s == NS - 1, N - WPS, s * WPS)
    for kk in range(WPS // ZR):
        pltpu.sync_copy(zbuf, acc.at[pl.ds(z + kk * ZR, ZR)])


def _write_out(acc, out_hbm, c, s):
    w = jnp.where(s == NS - 1, N - WPS, s * WPS)
    pltpu.sync_copy(acc.at[pl.ds(w, WPS)], out_hbm.at[c, pl.ds(w, WPS)])


def _pipelined_agg(issue_fetch, wait_fetch, reload_idx,
                   dst_v, rbuf, ssem, acc):
    reload_idx(0, 0)
    for j in range(K - 1):
        issue_fetch(j, j % K)

    def chunk(j, carry):
        g = j // IB
        jj = j - g * IB
        b = lax.rem(j, K)
        gb = lax.rem(g, 2)
        wait_fetch(j, b)
        pltpu.async_copy(rbuf.at[b], acc.at[dst_v.at[gb, jj]], ssem.at[b],
                         add=True)

        @pl.when(j + K - 1 < NCHUNK)
        def _():
            j1 = j + K - 1
            g1 = j1 // IB
            jj1 = j1 - g1 * IB
            nb = lax.rem(j1, K)
            g1b = lax.rem(g1, 2)

            @pl.when(jj1 == 0)
            def _():
                reload_idx(g1, g1b)

            @pl.when(j1 >= K)
            def _():
                pltpu.make_async_copy(rbuf.at[nb], acc.at[dst_v.at[g1b, jj1]],
                                      ssem.at[nb]).wait()
            issue_fetch(j1, nb)
        return carry
    lax.fori_loop(0, NCHUNK, chunk, 0)
    for r in range(K - 1):
        b = (NCHUNK - K + 1 + r) % K
        pltpu.make_async_copy(rbuf.at[b], acc.at[dst_v.at[0, 0]],
                              ssem.at[b]).wait()


@functools.partial(
    pl.kernel,
    out_type=_agg_out_type,
    mesh=_mesh,
    scratch_types=[
        pltpu.VMEM((2, IB, CH), jnp.int32),
        pltpu.VMEM((2, IB, CH), jnp.int32),
        pltpu.VMEM((K, CH, D), jnp.float32),
        pltpu.VMEM((ZR, D), jnp.float32),
        pltpu.SemaphoreType.DMA((K,)),
        pltpu.SemaphoreType.DMA((K,)),
        pltpu.VMEM_SHARED((ACC_R, D), jnp.float32),
    ],
)
def _sc_agg(tab_hbm, src_hbm, dst_hbm, out_hbm,
            src_v, dst_v, rbuf, zbuf, gsem, ssem, acc):
    c = lax.axis_index("c")
    s = lax.axis_index("s")
    _zero_acc(acc, zbuf, s)
    t = c * NS + s
    plsc.subcore_barrier()

    def issue_fetch(j, b):
        g = j // IB
        jj = j - g * IB
        gb = lax.rem(g, 2) if not isinstance(g, int) else g % 2
        pltpu.async_copy(tab_hbm.at[src_v.at[gb, jj]], rbuf.at[b],
                         gsem.at[b])

    def wait_fetch(j, b):
        g = j // IB
        jj = j - g * IB
        gb = lax.rem(g, 2) if not isinstance(g, int) else g % 2
        pltpu.make_async_copy(tab_hbm.at[src_v.at[gb, jj]], rbuf.at[b],
                              gsem.at[b]).wait()

    def reload_idx(g1, g1b):
        pltpu.sync_copy(src_hbm.at[t, g1], src_v.at[g1b])
        pltpu.sync_copy(dst_hbm.at[t, g1], dst_v.at[g1b])

    _pipelined_agg(issue_fetch, wait_fetch, reload_idx, dst_v, rbuf, ssem,
                   acc)
    plsc.subcore_barrier()
    _write_out(acc, out_hbm, c, s)


@functools.partial(
    pl.kernel,
    out_type=_agg_out_type,
    mesh=_mesh,
    scratch_types=[
        pltpu.VMEM((2, IB, CH), jnp.int32),
        pltpu.VMEM((K, CH, D), jnp.float32),
        pltpu.VMEM((ZR, D), jnp.float32),
        pltpu.SemaphoreType.DMA((K,)),
        pltpu.SemaphoreType.DMA((K,)),
        pltpu.VMEM_SHARED((ACC_R, D), jnp.float32),
    ],
)
def _sc_agg_linear(ea_hbm, dst_hbm, out_hbm,
                   dst_v, rbuf, zbuf, gsem, ssem, acc):
    c = lax.axis_index("c")
    s = lax.axis_index("s")
    _zero_acc(acc, zbuf, s)
    t = c * NS + s
    plsc.subcore_barrier()
    base = t * EPT

    def issue_fetch(j, b):
        pltpu.async_copy(ea_hbm.at[pl.ds(base + j * CH, CH)], rbuf.at[b],
                         gsem.at[b])

    def wait_fetch(j, b):
        pltpu.make_async_copy(ea_hbm.at[pl.ds(base + j * CH, CH)],
                              rbuf.at[b], gsem.at[b]).wait()

    def reload_idx(g1, g1b):
        pltpu.sync_copy(dst_hbm.at[t, g1], dst_v.at[g1b])

    _pipelined_agg(issue_fetch, wait_fetch, reload_idx, dst_v, rbuf, ssem,
                   acc)
    plsc.subcore_barrier()
    _write_out(acc, out_hbm, c, s)


def _gelu(x):
    return 0.5 * x * (1.0 + lax.erf(x * (1.0 / math.sqrt(2.0))))


def _ln(x, g, b):
    mu = jnp.mean(x, axis=-1, keepdims=True)
    var = jnp.mean((x - mu) ** 2, axis=-1, keepdims=True)
    return (x - mu) * lax.rsqrt(var + 1e-5) * g + b


def _mlp_body(a, e, h_ref, w1, b1, g1, bb1, w2, b2, g2, bb2, out_ref):
    h = h_ref[...]
    x = a.at[0][...] + a.at[1][...] + e.at[0][...] + e.at[1][...] + h
    u = jnp.dot(x, w1[...], preferred_element_type=jnp.float32) + b1[...]
    u = _gelu(_ln(u, g1[...], bb1[...]))
    v = jnp.dot(u, w2[...], preferred_element_type=jnp.float32) + b2[...]
    v = _ln(v, g2[...], bb2[...])
    out_ref[...] = _gelu(v + h)


_ROWS = 1000


def _tc_mlp(a, e, h, w1, b1, g1, bb1, w2, b2, g2, bb2):
    aspec = pl.BlockSpec((NC, _ROWS, D), lambda i: (0, i, 0))
    big = pl.BlockSpec((_ROWS, D), lambda i: (i, 0))
    wspec = pl.BlockSpec((D, D), lambda i: (0, 0))
    vspec = pl.BlockSpec((1, D), lambda i: (0, 0))
    return pl.pallas_call(
        _mlp_body,
        grid=(N // _ROWS,),
        in_specs=[aspec, aspec, big,
                  wspec, vspec, vspec, vspec,
                  wspec, vspec, vspec, vspec],
        out_specs=big,
        out_shape=jax.ShapeDtypeStruct((N, D), jnp.float32),
    )(a, e, h, w1, b1, g1, bb1, w2, b2, g2, bb2)


def kernel(h, batch, edge_index, h_edge_attr,
           W1, b1, ln1_g, ln1_b, W2, b2, ln2_g, ln2_b):
    del batch
    src = edge_index[0].astype(jnp.int32).reshape(TILES, NG, IB, CH)
    dst = edge_index[1].astype(jnp.int32).reshape(TILES, NG, IB, CH)

    e = _sc_agg_linear(h_edge_attr, dst)

    def body(x, ws):
        w1, bb1v, g1, bv1, w2, bb2v, g2, bv2 = ws
        a = _sc_agg(x, src, dst)
        x = _tc_mlp(a, e, x, w1, bb1v, g1, bv1, w2, bb2v, g2, bv2)
        return x, None

    ws = (W1, b1.reshape(L, 1, D), ln1_g.reshape(L, 1, D),
          ln1_b.reshape(L, 1, D), W2, b2.reshape(L, 1, D),
          ln2_g.reshape(L, 1, D), ln2_b.reshape(L, 1, D))
    x, _ = lax.scan(body, h, ws)
    return x


# speedup vs baseline: 1.0457x; 1.0457x over previous
import functools
import math

import jax
import jax.numpy as jnp
from jax import lax
from jax.experimental import pallas as pl
from jax.experimental.pallas import tpu as pltpu
from jax.experimental.pallas import tpu_sc as plsc

N = 10000
E = 320000
D = 128
L = 3

NC = 2
NS = 16
TILES = NC * NS
EPT = E // TILES
CH = 40
NCHUNK = EPT // CH
IB = 25
NG = NCHUNK // IB
K = 6
ACC_R = N
WPS = 640
ZR = 16

_mesh = plsc.VectorSubcoreMesh(
    core_axis_name="c", subcore_axis_name="s", num_cores=NC, num_subcores=NS)

_agg_out_type = jax.ShapeDtypeStruct((NC, N, D), jnp.float32)


def _zero_acc(acc, zbuf, zsem, s):
    def zstore(t, carry):
        i = t // (D // 16)
        k = t % (D // 16)
        zbuf[i, pl.ds(k * 16, 16)] = jnp.zeros((16,), jnp.float32)
        return carry
    lax.fori_loop(0, ZR * (D // 16), zstore, 0)
    z = jnp.where(s == NS - 1, N - WPS, s * WPS)
    for kk in range(WPS // ZR):
        pltpu.async_copy(zbuf, acc.at[pl.ds(z + kk * ZR, ZR)], zsem)
    for kk in range(WPS // ZR):
        pltpu.make_async_copy(zbuf, acc.at[pl.ds(z + kk * ZR, ZR)],
                              zsem).wait()


def _write_out(acc, out_hbm, c, s):
    w = jnp.where(s == NS - 1, N - WPS, s * WPS)
    pltpu.sync_copy(acc.at[pl.ds(w, WPS)], out_hbm.at[c, pl.ds(w, WPS)])


def _pipelined_agg(issue_fetch, wait_fetch, reload_idx,
                   dst_v, rbuf, ssem, acc):
    def chunk(j, carry):
        g = j // IB
        jj = j - g * IB
        b = lax.rem(j, K)
        gb = lax.rem(g, 2)
        wait_fetch(j, b)
        pltpu.async_copy(rbuf.at[b], acc.at[dst_v.at[gb, jj]], ssem.at[b],
                         add=True)

        @pl.when(j + K - 1 < NCHUNK)
        def _():
            j1 = j + K - 1
            g1 = j1 // IB
            jj1 = j1 - g1 * IB
            nb = lax.rem(j1, K)
            g1b = lax.rem(g1, 2)

            @pl.when(jj1 == 0)
            def _():
                reload_idx(g1, g1b)

            @pl.when(j1 >= K)
            def _():
                pltpu.make_async_copy(rbuf.at[nb], acc.at[dst_v.at[g1b, jj1]],
                                      ssem.at[nb]).wait()
            issue_fetch(j1, nb)
        return carry
    lax.fori_loop(0, NCHUNK, chunk, 0)
    for r in range(K - 1):
        b = (NCHUNK - K + 1 + r) % K
        pltpu.make_async_copy(rbuf.at[b], acc.at[dst_v.at[0, 0]],
                              ssem.at[b]).wait()


@functools.partial(
    pl.kernel,
    out_type=_agg_out_type,
    mesh=_mesh,
    scratch_types=[
        pltpu.VMEM((2, IB, CH), jnp.int32),
        pltpu.VMEM((2, IB, CH), jnp.int32),
        pltpu.VMEM((K, CH, D), jnp.float32),
        pltpu.VMEM((ZR, D), jnp.float32),
        pltpu.SemaphoreType.DMA((K,)),
        pltpu.SemaphoreType.DMA((K,)),
        pltpu.SemaphoreType.DMA,
        pltpu.VMEM_SHARED((ACC_R, D), jnp.float32),
    ],
)
def _sc_agg(tab_hbm, src_hbm, dst_hbm, out_hbm,
            src_v, dst_v, rbuf, zbuf, gsem, ssem, zsem, acc):
    c = lax.axis_index("c")
    s = lax.axis_index("s")
    t = c * NS + s

    def issue_fetch(j, b):
        g = j // IB
        jj = j - g * IB
        gb = lax.rem(g, 2) if not isinstance(g, int) else g % 2
        pltpu.async_copy(tab_hbm.at[src_v.at[gb, jj]], rbuf.at[b],
                         gsem.at[b])

    def wait_fetch(j, b):
        g = j // IB
        jj = j - g * IB
        gb = lax.rem(g, 2) if not isinstance(g, int) else g % 2
        pltpu.make_async_copy(tab_hbm.at[src_v.at[gb, jj]], rbuf.at[b],
                              gsem.at[b]).wait()

    def reload_idx(g1, g1b):
        pltpu.sync_copy(src_hbm.at[t, g1], src_v.at[g1b])
        pltpu.sync_copy(dst_hbm.at[t, g1], dst_v.at[g1b])

    reload_idx(0, 0)
    for j in range(K - 1):
        issue_fetch(j, j % K)
    _zero_acc(acc, zbuf, zsem, s)
    plsc.subcore_barrier()
    _pipelined_agg(issue_fetch, wait_fetch, reload_idx, dst_v, rbuf, ssem,
                   acc)
    plsc.subcore_barrier()
    _write_out(acc, out_hbm, c, s)


@functools.partial(
    pl.kernel,
    out_type=_agg_out_type,
    mesh=_mesh,
    scratch_types=[
        pltpu.VMEM((2, IB, CH), jnp.int32),
        pltpu.VMEM((K, CH, D), jnp.float32),
        pltpu.VMEM((ZR, D), jnp.float32),
        pltpu.SemaphoreType.DMA((K,)),
        pltpu.SemaphoreType.DMA((K,)),
        pltpu.SemaphoreType.DMA,
        pltpu.VMEM_SHARED((ACC_R, D), jnp.float32),
    ],
)
def _sc_agg_linear(ea_hbm, dst_hbm, out_hbm,
                   dst_v, rbuf, zbuf, gsem, ssem, zsem, acc):
    c = lax.axis_index("c")
    s = lax.axis_index("s")
    t = c * NS + s
    base = t * EPT

    def issue_fetch(j, b):
        pltpu.async_copy(ea_hbm.at[pl.ds(base + j * CH, CH)], rbuf.at[b],
                         gsem.at[b])

    def wait_fetch(j, b):
        pltpu.make_async_copy(ea_hbm.at[pl.ds(base + j * CH, CH)],
                              rbuf.at[b], gsem.at[b]).wait()

    def reload_idx(g1, g1b):
        pltpu.sync_copy(dst_hbm.at[t, g1], dst_v.at[g1b])

    reload_idx(0, 0)
    for j in range(K - 1):
        issue_fetch(j, j % K)
    _zero_acc(acc, zbuf, zsem, s)
    plsc.subcore_barrier()
    _pipelined_agg(issue_fetch, wait_fetch, reload_idx, dst_v, rbuf, ssem,
                   acc)
    plsc.subcore_barrier()
    _write_out(acc, out_hbm, c, s)


def _gelu(x):
    return 0.5 * x * (1.0 + lax.erf(x * (1.0 / math.sqrt(2.0))))


def _ln(x, g, b):
    mu = jnp.mean(x, axis=-1, keepdims=True)
    var = jnp.mean((x - mu) ** 2, axis=-1, keepdims=True)
    return (x - mu) * lax.rsqrt(var + 1e-5) * g + b


def _mlp_body(a, e, h_ref, w1, b1, g1, bb1, w2, b2, g2, bb2, out_ref):
    h = h_ref[...]
    x = a.at[0][...] + a.at[1][...] + e.at[0][...] + e.at[1][...] + h
    u = jnp.dot(x, w1[...], preferred_element_type=jnp.float32) + b1[...]
    u = _gelu(_ln(u, g1[...], bb1[...]))
    v = jnp.dot(u, w2[...], preferred_element_type=jnp.float32) + b2[...]
    v = _ln(v, g2[...], bb2[...])
    out_ref[...] = _gelu(v + h)


_ROWS = 1000


def _tc_mlp(a, e, h, w1, b1, g1, bb1, w2, b2, g2, bb2):
    aspec = pl.BlockSpec((NC, _ROWS, D), lambda i: (0, i, 0))
    big = pl.BlockSpec((_ROWS, D), lambda i: (i, 0))
    wspec = pl.BlockSpec((D, D), lambda i: (0, 0))
    vspec = pl.BlockSpec((1, D), lambda i: (0, 0))
    return pl.pallas_call(
        _mlp_body,
        grid=(N // _ROWS,),
        in_specs=[aspec, aspec, big,
                  wspec, vspec, vspec, vspec,
                  wspec, vspec, vspec, vspec],
        out_specs=big,
        out_shape=jax.ShapeDtypeStruct((N, D), jnp.float32),
    )(a, e, h, w1, b1, g1, bb1, w2, b2, g2, bb2)


def kernel(h, batch, edge_index, h_edge_attr,
           W1, b1, ln1_g, ln1_b, W2, b2, ln2_g, ln2_b):
    del batch
    src = edge_index[0].astype(jnp.int32).reshape(TILES, NG, IB, CH)
    dst = edge_index[1].astype(jnp.int32).reshape(TILES, NG, IB, CH)

    e = _sc_agg_linear(h_edge_attr, dst)

    def body(x, ws):
        w1, bb1v, g1, bv1, w2, bb2v, g2, bv2 = ws
        a = _sc_agg(x, src, dst)
        x = _tc_mlp(a, e, x, w1, bb1v, g1, bv1, w2, bb2v, g2, bv2)
        return x, None

    ws = (W1, b1.reshape(L, 1, D), ln1_g.reshape(L, 1, D),
          ln1_b.reshape(L, 1, D), W2, b2.reshape(L, 1, D),
          ln2_g.reshape(L, 1, D), ln2_b.reshape(L, 1, D))
    x, _ = lax.scan(body, h, ws)
    return x


# speedup vs baseline: 1.1093x; 1.0608x over previous
import functools
import math

import jax
import jax.numpy as jnp
from jax import lax
from jax.experimental import pallas as pl
from jax.experimental.pallas import tpu as pltpu
from jax.experimental.pallas import tpu_sc as plsc

N = 10000
E = 320000
D = 128
L = 3

NC = 2
NS = 16
TILES = NC * NS
EPT = E // TILES
CH = 40
NCHUNK = EPT // CH
IB = 25
NG = NCHUNK // IB
K = 6
PREF = 10
ACC_R = N
WPS = 640
ZR = 16

_mesh = plsc.VectorSubcoreMesh(
    core_axis_name="c", subcore_axis_name="s", num_cores=NC, num_subcores=NS)

_agg_out_type = jax.ShapeDtypeStruct((NC, N, D), jnp.float32)


def _zero_acc(acc, zbuf, zsem, s):
    def zstore(t, carry):
        i = t // (D // 16)
        k = t % (D // 16)
        zbuf[i, pl.ds(k * 16, 16)] = jnp.zeros((16,), jnp.float32)
        return carry
    lax.fori_loop(0, ZR * (D // 16), zstore, 0)
    z = jnp.where(s == NS - 1, N - WPS, s * WPS)
    for kk in range(WPS // ZR):
        pltpu.async_copy(zbuf, acc.at[pl.ds(z + kk * ZR, ZR)], zsem)
    for kk in range(WPS // ZR):
        pltpu.make_async_copy(zbuf, acc.at[pl.ds(z + kk * ZR, ZR)],
                              zsem).wait()


def _write_out(acc, out_hbm, c, s):
    w = jnp.where(s == NS - 1, N - WPS, s * WPS)
    pltpu.sync_copy(acc.at[pl.ds(w, WPS)], out_hbm.at[c, pl.ds(w, WPS)])


def _pipelined_agg(issue_fetch, wait_fetch, issue_reload, wait_reload,
                   dst_v, rbuf, ssem, acc):
    def chunk(j, carry):
        g = j // IB
        jj = j - g * IB
        b = lax.rem(j, K)
        gb = lax.rem(g, 2)
        wait_fetch(j, b)
        pltpu.async_copy(rbuf.at[b], acc.at[dst_v.at[gb, jj]], ssem.at[b],
                         add=True)

        @pl.when(j + K - 1 < NCHUNK)
        def _():
            j1 = j + K - 1
            g1 = j1 // IB
            jj1 = j1 - g1 * IB
            nb = lax.rem(j1, K)
            g1b = lax.rem(g1, 2)

            @pl.when(jj1 == 0)
            def _():
                wait_reload(g1, g1b)

            @pl.when((jj1 == PREF) & (j1 < (NG - 1) * IB))
            def _():
                issue_reload(g1 + 1, lax.rem(g1 + 1, 2))

            @pl.when(j1 >= K)
            def _():
                pltpu.make_async_copy(rbuf.at[nb], acc.at[dst_v.at[g1b, jj1]],
                                      ssem.at[nb]).wait()
            issue_fetch(j1, nb)
        return carry
    lax.fori_loop(0, NCHUNK, chunk, 0)
    for r in range(K - 1):
        b = (NCHUNK - K + 1 + r) % K
        pltpu.make_async_copy(rbuf.at[b], acc.at[dst_v.at[0, 0]],
                              ssem.at[b]).wait()


@functools.partial(
    pl.kernel,
    out_type=_agg_out_type,
    mesh=_mesh,
    scratch_types=[
        pltpu.VMEM((2, IB, CH), jnp.int32),
        pltpu.VMEM((2, IB, CH), jnp.int32),
        pltpu.VMEM((K, CH, D), jnp.float32),
        pltpu.VMEM((ZR, D), jnp.float32),
        pltpu.SemaphoreType.DMA((K,)),
        pltpu.SemaphoreType.DMA((K,)),
        pltpu.SemaphoreType.DMA,
        pltpu.SemaphoreType.DMA,
        pltpu.VMEM_SHARED((ACC_R, D), jnp.float32),
    ],
)
def _sc_agg(tab_hbm, src_hbm, dst_hbm, out_hbm,
            src_v, dst_v, rbuf, zbuf, gsem, ssem, zsem, rsem, acc):
    c = lax.axis_index("c")
    s = lax.axis_index("s")
    t = c * NS + s

    def issue_fetch(j, b):
        g = j // IB
        jj = j - g * IB
        gb = lax.rem(g, 2) if not isinstance(g, int) else g % 2
        pltpu.async_copy(tab_hbm.at[src_v.at[gb, jj]], rbuf.at[b],
                         gsem.at[b])

    def wait_fetch(j, b):
        g = j // IB
        jj = j - g * IB
        gb = lax.rem(g, 2) if not isinstance(g, int) else g % 2
        pltpu.make_async_copy(tab_hbm.at[src_v.at[gb, jj]], rbuf.at[b],
                              gsem.at[b]).wait()

    def issue_reload(g1, g1b):
        pltpu.async_copy(src_hbm.at[t, g1], src_v.at[g1b], rsem)
        pltpu.async_copy(dst_hbm.at[t, g1], dst_v.at[g1b], rsem)

    def wait_reload(g1, g1b):
        pltpu.make_async_copy(src_hbm.at[t, g1], src_v.at[g1b],
                              rsem).wait()
        pltpu.make_async_copy(dst_hbm.at[t, g1], dst_v.at[g1b],
                              rsem).wait()

    issue_reload(0, 0)
    wait_reload(0, 0)
    for j in range(K - 1):
        issue_fetch(j, j % K)
    _zero_acc(acc, zbuf, zsem, s)
    plsc.subcore_barrier()
    _pipelined_agg(issue_fetch, wait_fetch, issue_reload, wait_reload,
                   dst_v, rbuf, ssem, acc)
    plsc.subcore_barrier()
    _write_out(acc, out_hbm, c, s)


@functools.partial(
    pl.kernel,
    out_type=_agg_out_type,
    mesh=_mesh,
    scratch_types=[
        pltpu.VMEM((2, IB, CH), jnp.int32),
        pltpu.VMEM((K, CH, D), jnp.float32),
        pltpu.VMEM((ZR, D), jnp.float32),
        pltpu.SemaphoreType.DMA((K,)),
        pltpu.SemaphoreType.DMA((K,)),
        pltpu.SemaphoreType.DMA,
        pltpu.SemaphoreType.DMA,
        pltpu.VMEM_SHARED((ACC_R, D), jnp.float32),
    ],
)
def _sc_agg_linear(ea_hbm, dst_hbm, out_hbm,
                   dst_v, rbuf, zbuf, gsem, ssem, zsem, rsem, acc):
    c = lax.axis_index("c")
    s = lax.axis_index("s")
    t = c * NS + s
    base = t * EPT

    def issue_fetch(j, b):
        pltpu.async_copy(ea_hbm.at[pl.ds(base + j * CH, CH)], rbuf.at[b],
                         gsem.at[b])

    def wait_fetch(j, b):
        pltpu.make_async_copy(ea_hbm.at[pl.ds(base + j * CH, CH)],
                              rbuf.at[b], gsem.at[b]).wait()

    def issue_reload(g1, g1b):
        pltpu.async_copy(dst_hbm.at[t, g1], dst_v.at[g1b], rsem)

    def wait_reload(g1, g1b):
        pltpu.make_async_copy(dst_hbm.at[t, g1], dst_v.at[g1b],
                              rsem).wait()

    issue_reload(0, 0)
    wait_reload(0, 0)
    for j in range(K - 1):
        issue_fetch(j, j % K)
    _zero_acc(acc, zbuf, zsem, s)
    plsc.subcore_barrier()
    _pipelined_agg(issue_fetch, wait_fetch, issue_reload, wait_reload,
                   dst_v, rbuf, ssem, acc)
    plsc.subcore_barrier()
    _write_out(acc, out_hbm, c, s)


def _gelu(x):
    return 0.5 * x * (1.0 + lax.erf(x * (1.0 / math.sqrt(2.0))))


def _ln(x, g, b):
    mu = jnp.mean(x, axis=-1, keepdims=True)
    var = jnp.mean((x - mu) ** 2, axis=-1, keepdims=True)
    return (x - mu) * lax.rsqrt(var + 1e-5) * g + b


def _mlp_body(a, e, h_ref, w1, b1, g1, bb1, w2, b2, g2, bb2, out_ref):
    h = h_ref[...]
    x = a.at[0][...] + a.at[1][...] + e.at[0][...] + e.at[1][...] + h
    u = jnp.dot(x, w1[...], preferred_element_type=jnp.float32) + b1[...]
    u = _gelu(_ln(u, g1[...], bb1[...]))
    v = jnp.dot(u, w2[...], preferred_element_type=jnp.float32) + b2[...]
    v = _ln(v, g2[...], bb2[...])
    out_ref[...] = _gelu(v + h)


_ROWS = 1000


def _tc_mlp(a, e, h, w1, b1, g1, bb1, w2, b2, g2, bb2):
    aspec = pl.BlockSpec((NC, _ROWS, D), lambda i: (0, i, 0))
    big = pl.BlockSpec((_ROWS, D), lambda i: (i, 0))
    wspec = pl.BlockSpec((D, D), lambda i: (0, 0))
    vspec = pl.BlockSpec((1, D), lambda i: (0, 0))
    return pl.pallas_call(
        _mlp_body,
        grid=(N // _ROWS,),
        in_specs=[aspec, aspec, big,
                  wspec, vspec, vspec, vspec,
                  wspec, vspec, vspec, vspec],
        out_specs=big,
        out_shape=jax.ShapeDtypeStruct((N, D), jnp.float32),
    )(a, e, h, w1, b1, g1, bb1, w2, b2, g2, bb2)


def kernel(h, batch, edge_index, h_edge_attr,
           W1, b1, ln1_g, ln1_b, W2, b2, ln2_g, ln2_b):
    del batch
    src = edge_index[0].astype(jnp.int32).reshape(TILES, NG, IB, CH)
    dst = edge_index[1].astype(jnp.int32).reshape(TILES, NG, IB, CH)

    e = _sc_agg_linear(h_edge_attr, dst)

    def body(x, ws):
        w1, bb1v, g1, bv1, w2, bb2v, g2, bv2 = ws
        a = _sc_agg(x, src, dst)
        x = _tc_mlp(a, e, x, w1, bb1v, g1, bv1, w2, bb2v, g2, bv2)
        return x, None

    ws = (W1, b1.reshape(L, 1, D), ln1_g.reshape(L, 1, D),
          ln1_b.reshape(L, 1, D), W2, b2.reshape(L, 1, D),
          ln2_g.reshape(L, 1, D), ln2_b.reshape(L, 1, D))
    x, _ = lax.scan(body, h, ws)
    return x
